# Initial kernel scaffold; baseline (speedup 1.0000x reference)
import jax
import jax.numpy as jnp
from jax.experimental import pallas as pl


def kernel(x, W1l, b1l, W1r, b1r, att1, bias1, W2l, b2l, W2r, b2r, att2, bias2, edge_index, batch):
    raise NotImplementedError("write your pallas kernel here")


# speedup vs baseline: 14.7090x; 14.7090x over previous
import functools

import jax
import jax.numpy as jnp
from jax import lax
from jax.experimental import pallas as pl
from jax.experimental.pallas import tpu as pltpu
from jax.experimental.pallas import tpu_sc as plsc

F32 = jnp.float32
I32 = jnp.int32

CH = 128
NTILES = 32


def _cdiv(a, b):
    return (a + b - 1) // b


def _tc_node_transform(x, Wl, bl, Wr, br, nb):
    n, f = x.shape
    k = Wl.shape[1]
    grid = (n // nb,)

    def body(x_ref, wl_ref, bl_ref, wr_ref, br_ref, ol_ref, or_ref):
        xb = x_ref[...]
        ol_ref[...] = jnp.dot(xb, wl_ref[...], preferred_element_type=F32) + bl_ref[...]
        or_ref[...] = jnp.dot(xb, wr_ref[...], preferred_element_type=F32) + br_ref[...]

    return pl.pallas_call(
        body,
        grid=grid,
        in_specs=[
            pl.BlockSpec((nb, f), lambda i: (i, 0)),
            pl.BlockSpec((f, k), lambda i: (0, 0)),
            pl.BlockSpec((1, k), lambda i: (0, 0)),
            pl.BlockSpec((f, k), lambda i: (0, 0)),
            pl.BlockSpec((1, k), lambda i: (0, 0)),
        ],
        out_specs=[pl.BlockSpec((nb, k), lambda i: (i, 0)),
                   pl.BlockSpec((nb, k), lambda i: (i, 0))],
        out_shape=[jax.ShapeDtypeStruct((n, k), F32),
                   jax.ShapeDtypeStruct((n, k), F32)],
    )(x, Wl, bl.reshape(1, k), Wr, br.reshape(1, k))


def _tc_edge_scores(gl, gr, amat, eb):
    epad, k = gl.shape
    h = amat.shape[1]
    grid = (epad // eb,)

    def body(gl_ref, gr_ref, a_ref, e_ref, m_ref):
        z = gl_ref[...] + gr_ref[...]
        z = jnp.where(z > 0, z, 0.2 * z)
        e = jnp.dot(z, a_ref[...], preferred_element_type=F32)
        e_ref[...] = e
        bm = jnp.max(e, axis=0, keepdims=True)

        @pl.when(pl.program_id(0) == 0)
        def _():
            m_ref[...] = jnp.full((1, h), -jnp.inf, F32)

        m_ref[...] = jnp.maximum(m_ref[...], bm)

    return pl.pallas_call(
        body,
        grid=grid,
        in_specs=[
            pl.BlockSpec((eb, k), lambda i: (i, 0)),
            pl.BlockSpec((eb, k), lambda i: (i, 0)),
            pl.BlockSpec((k, h), lambda i: (0, 0)),
        ],
        out_specs=[pl.BlockSpec((eb, h), lambda i: (i, 0)),
                   pl.BlockSpec((1, h), lambda i: (0, 0))],
        out_shape=[jax.ShapeDtypeStruct((epad, h), F32),
                   jax.ShapeDtypeStruct((1, h), F32)],
    )(gl, gr, amat)


def _tc_edge_weight(gl, e, m, rep, r16, eb):
    epad, k = gl.shape
    h = e.shape[1]
    grid = (epad // eb,)

    def body(gl_ref, e_ref, m_ref, rep_ref, r16_ref, w_ref, x_ref):
        ex = jnp.exp(e_ref[...] - m_ref[...])
        w_ref[...] = gl_ref[...] * jnp.dot(ex, rep_ref[...], preferred_element_type=F32)
        x_ref[...] = jnp.dot(ex, r16_ref[...], preferred_element_type=F32)

    return pl.pallas_call(
        body,
        grid=grid,
        in_specs=[
            pl.BlockSpec((eb, k), lambda i: (i, 0)),
            pl.BlockSpec((eb, h), lambda i: (i, 0)),
            pl.BlockSpec((1, h), lambda i: (0, 0)),
            pl.BlockSpec((h, k), lambda i: (0, 0)),
            pl.BlockSpec((h, 16), lambda i: (0, 0)),
        ],
        out_specs=[pl.BlockSpec((eb, k), lambda i: (i, 0)),
                   pl.BlockSpec((eb, 16), lambda i: (i, 0))],
        out_shape=[jax.ShapeDtypeStruct((epad, k), F32),
                   jax.ShapeDtypeStruct((epad, 16), F32)],
    )(gl, e, m, rep, r16)


def _tc_norm_elu_transform(acc, s, bias, rep, W2l, b2l, W2r, b2r, nb):
    npad, k = acc.shape
    h4 = rep.shape[0]
    d2 = W2l.shape[1]
    grid = (npad // nb,)

    def body(acc_ref, s_ref, bias_ref, rep_ref, wl_ref, bl_ref, wr_ref, br_ref,
             ol_ref, or_ref):
        s4 = s_ref[...][:, :h4]
        recip = 1.0 / (s4 + 1e-16)
        hv = acc_ref[...] * jnp.dot(recip, rep_ref[...], preferred_element_type=F32)
        hv = hv + bias_ref[...]
        hv = jnp.where(hv > 0, hv, jnp.exp(hv) - 1.0)
        ol_ref[...] = jnp.dot(hv, wl_ref[...], preferred_element_type=F32) + bl_ref[...]
        or_ref[...] = jnp.dot(hv, wr_ref[...], preferred_element_type=F32) + br_ref[...]

    return pl.pallas_call(
        body,
        grid=grid,
        in_specs=[
            pl.BlockSpec((nb, k), lambda i: (i, 0)),
            pl.BlockSpec((nb, 16), lambda i: (i, 0)),
            pl.BlockSpec((1, k), lambda i: (0, 0)),
            pl.BlockSpec((h4, k), lambda i: (0, 0)),
            pl.BlockSpec((k, d2), lambda i: (0, 0)),
            pl.BlockSpec((1, d2), lambda i: (0, 0)),
            pl.BlockSpec((k, d2), lambda i: (0, 0)),
            pl.BlockSpec((1, d2), lambda i: (0, 0)),
        ],
        out_specs=[pl.BlockSpec((nb, d2), lambda i: (i, 0)),
                   pl.BlockSpec((nb, d2), lambda i: (i, 0))],
        out_shape=[jax.ShapeDtypeStruct((npad, d2), F32),
                   jax.ShapeDtypeStruct((npad, d2), F32)],
    )(acc, s, bias.reshape(1, k), rep, W2l, b2l.reshape(1, d2), W2r,
      b2r.reshape(1, d2))


def _tc_final_pool(acc, s, bias, batch3, n, g, nb):
    npad, d = acc.shape
    grid = (n // nb,)
    nsteps = n // nb

    def body(acc_ref, s_ref, bias_ref, b_ref, out_ref, sums_ref, cnts_ref):
        i = pl.program_id(0)
        s0 = s_ref[...][:, :1]
        hv = acc_ref[...] / (s0 + 1e-16) + bias_ref[...]
        hv = jnp.where(hv > 0, hv, jnp.exp(hv) - 1.0)
        bq = b_ref[...].reshape(1, nb)
        gids = lax.broadcasted_iota(I32, (g, 1), 0)
        oh = (bq == gids).astype(F32)
        part = lax.dot_general(oh, hv, (((1,), (0,)), ((), ())),
                               preferred_element_type=F32)
        pc = jnp.sum(oh, axis=1, keepdims=True)

        @pl.when(i == 0)
        def _():
            sums_ref[...] = jnp.zeros_like(sums_ref)
            cnts_ref[...] = jnp.zeros_like(cnts_ref)

        sums_ref[...] += part
        cnts_ref[...] += jnp.broadcast_to(pc, cnts_ref.shape)

        @pl.when(i == nsteps - 1)
        def _():
            out_ref[...] = sums_ref[...] / jnp.maximum(cnts_ref[...], 1.0)

    return pl.pallas_call(
        body,
        grid=grid,
        in_specs=[
            pl.BlockSpec((nb, d), lambda i: (i, 0)),
            pl.BlockSpec((nb, 16), lambda i: (i, 0)),
            pl.BlockSpec((1, d), lambda i: (0, 0)),
            pl.BlockSpec((1, 1, nb), lambda i: (i, 0, 0)),
        ],
        out_specs=pl.BlockSpec((g, d), lambda i: (0, 0)),
        out_shape=jax.ShapeDtypeStruct((g, d), F32),
        scratch_shapes=[pltpu.VMEM((g, d), F32), pltpu.VMEM((g, d), F32)],
    )(acc, s, bias.reshape(1, d), batch3)


def _sc_gather(xl, xr, srcp, dstp):
    epad = srcp.shape[0]
    dw = xl.shape[1]
    span = epad // NTILES
    iters = span // CH
    mesh = plsc.VectorSubcoreMesh(core_axis_name="c", subcore_axis_name="s")

    @functools.partial(
        pl.kernel,
        out_type=(jax.ShapeDtypeStruct((epad, dw), F32),
                  jax.ShapeDtypeStruct((epad, dw), F32)),
        mesh=mesh,
        scratch_types=[
            pltpu.VMEM((CH,), I32), pltpu.VMEM((CH,), I32),
            pltpu.VMEM((CH, dw), F32), pltpu.VMEM((CH, dw), F32),
            pltpu.SemaphoreType.DMA, pltpu.SemaphoreType.DMA,
        ],
        compiler_params=pltpu.CompilerParams(use_tc_tiling_on_sc=False),
    )
    def k(xl_h, xr_h, src_h, dst_h, gl_h, gr_h, sib, dib, rowl, rowr, sm1, sm2):
        c = lax.axis_index("c")
        t = lax.axis_index("s")
        wid = t * 2 + c

        def chunk(j, carry):
            base = wid * span + j * CH
            pltpu.sync_copy(src_h.at[pl.ds(base, CH)], sib)
            pltpu.sync_copy(dst_h.at[pl.ds(base, CH)], dib)
            cp1 = pltpu.async_copy(xl_h.at[sib], rowl, sm1)
            cp2 = pltpu.async_copy(xr_h.at[dib], rowr, sm2)
            cp1.wait()
            cp2.wait()
            pltpu.sync_copy(rowl, gl_h.at[pl.ds(base, CH)])
            pltpu.sync_copy(rowr, gr_h.at[pl.ds(base, CH)])
            return carry

        lax.fori_loop(0, iters, chunk, 0)

    return k(xl, xr, srcp, dstp)


def _sc_scatter(rows, dstp, z_a, e_real, rng, nrounds):
    epad = dstp.shape[0]
    dw = rows.shape[1]
    half = nrounds * rng
    npad = 2 * half
    span = epad // 16
    iters = span // CH
    za = (rng + 16) // 16
    ca = rng // 16
    mesh = plsc.VectorSubcoreMesh(core_axis_name="c", subcore_axis_name="s")

    @functools.partial(
        pl.kernel,
        out_type=jax.ShapeDtypeStruct((npad, dw), F32),
        mesh=mesh,
        scratch_types=[
            pltpu.VMEM_SHARED((rng + 16, dw), F32),
            pltpu.VMEM((CH,), I32),
            pltpu.VMEM((CH, dw), F32),
            pltpu.VMEM((CH,), I32),
        ],
        compiler_params=pltpu.CompilerParams(use_tc_tiling_on_sc=False),
    )
    def k(w_h, dst_h, za_h, acc_h, accslab, dstb, rowb, idxa):
        c = lax.axis_index("c")
        t = lax.axis_index("s")
        for r in range(nrounds):
            lo = c * half + r * rng
            pltpu.sync_copy(za_h.at[pl.ds(0, za)], accslab.at[pl.ds(t * za, za)])
            plsc.subcore_barrier()

            def chunk(j, carry):
                base = t * span + j * CH
                pltpu.sync_copy(dst_h.at[pl.ds(base, CH)], dstb)
                pltpu.sync_copy(w_h.at[pl.ds(base, CH)], rowb)
                for gi in range(CH // 16):
                    d = dstb[pl.ds(gi * 16, 16)]
                    pos = base + gi * 16 + lax.broadcasted_iota(I32, (16,), 0)
                    pad = pos >= e_real
                    rela = d - lo
                    oob = (rela < 0) | (rela >= rng) | pad
                    idxa[pl.ds(gi * 16, 16)] = jnp.where(oob, rng + gi, rela)
                pltpu.sync_copy(rowb, accslab.at[idxa], add=True)
                return carry

            lax.fori_loop(0, iters, chunk, 0)
            plsc.subcore_barrier()
            pltpu.sync_copy(accslab.at[pl.ds(t * ca, ca)],
                            acc_h.at[pl.ds(lo + t * ca, ca)])
            if r != nrounds - 1:
                plsc.subcore_barrier()

    return k(rows, dstp, z_a)


def kernel(x, W1l, b1l, W1r, b1r, att1, bias1, W2l, b2l, W2r, b2r, att2,
           bias2, edge_index, batch):
    n, f_in = x.shape
    e = edge_index.shape[1]
    h1, d = att1.shape
    hd = h1 * d
    g = 64

    epad = _cdiv(e, NTILES * CH) * NTILES * CH
    rng1 = _cdiv(n, 4 * 256) * 256
    half = 2 * rng1
    npad = 2 * half

    src = edge_index[0]
    dst = edge_index[1]
    padlen = epad - e
    srcp = jnp.concatenate([src, jnp.zeros((padlen,), I32)])
    dstp = jnp.concatenate([dst, jnp.zeros((padlen,), I32)])
    batch3 = batch.reshape(25, 1, n // 25)

    a1 = jnp.einsum("hd,hk->hdk", att1, jnp.eye(h1, dtype=F32)).reshape(hd, h1)
    rep1 = jnp.kron(jnp.eye(h1, dtype=F32), jnp.ones((1, d), F32))
    r16a = jnp.eye(h1, 16, dtype=F32)
    a2 = jnp.zeros((d, 8), F32).at[:, 0].set(att2[0])
    rep2 = jnp.zeros((8, d), F32).at[0, :].set(1.0)
    r16b = jnp.zeros((8, 16), F32).at[0, 0].set(1.0)

    z_a1 = jnp.zeros(((rng1 + 16) // 16, hd), F32)
    z_s1 = jnp.zeros(((half + 16) // 16, 16), F32)
    z_a2 = jnp.zeros(((half + 16) // 16, d), F32)

    xl1, xr1 = _tc_node_transform(x, W1l, b1l, W1r, b1r, nb=2000)
    gl1, gr1 = _sc_gather(xl1, xr1, srcp, dstp)
    e1, m1 = _tc_edge_scores(gl1, gr1, a1, eb=4096)
    wmsg1, exv1 = _tc_edge_weight(gl1, e1, m1, rep1, r16a, eb=4096)
    acc1 = _sc_scatter(wmsg1, dstp, z_a1, e, rng1, nrounds=2)
    s1 = _sc_scatter(exv1, dstp, z_s1, e, half, nrounds=1)
    xl2, xr2 = _tc_norm_elu_transform(acc1, s1, bias1, rep1, W2l, b2l, W2r,
                                      b2r, nb=1792)

    gl2, gr2 = _sc_gather(xl2, xr2, srcp, dstp)
    e2, m2 = _tc_edge_scores(gl2, gr2, a2, eb=4096)
    wmsg2, exv2 = _tc_edge_weight(gl2, e2, m2, rep2, r16b, eb=4096)
    acc2 = _sc_scatter(wmsg2, dstp, z_a2, e, half, nrounds=1)
    s2 = _sc_scatter(exv2, dstp, z_s1, e, half, nrounds=1)

    return _tc_final_pool(acc2, s2, bias2, batch3, n, g, nb=2000)


# speedup vs baseline: 17.9497x; 1.2203x over previous
import functools

import jax
import jax.numpy as jnp
from jax import lax
from jax.experimental import pallas as pl
from jax.experimental.pallas import tpu as pltpu
from jax.experimental.pallas import tpu_sc as plsc

F32 = jnp.float32
I32 = jnp.int32

CH = 128
NTILES = 32


def _cdiv(a, b):
    return (a + b - 1) // b


def _tc_node_transform(x, Wl, bl, Wr, br, nb):
    n, f = x.shape
    k = Wl.shape[1]
    grid = (n // nb,)

    def body(x_ref, wl_ref, bl_ref, wr_ref, br_ref, ol_ref, or_ref):
        xb = x_ref[...]
        ol_ref[...] = jnp.dot(xb, wl_ref[...], preferred_element_type=F32) + bl_ref[...]
        or_ref[...] = jnp.dot(xb, wr_ref[...], preferred_element_type=F32) + br_ref[...]

    return pl.pallas_call(
        body,
        grid=grid,
        in_specs=[
            pl.BlockSpec((nb, f), lambda i: (i, 0)),
            pl.BlockSpec((f, k), lambda i: (0, 0)),
            pl.BlockSpec((1, k), lambda i: (0, 0)),
            pl.BlockSpec((f, k), lambda i: (0, 0)),
            pl.BlockSpec((1, k), lambda i: (0, 0)),
        ],
        out_specs=[pl.BlockSpec((nb, k), lambda i: (i, 0)),
                   pl.BlockSpec((nb, k), lambda i: (i, 0))],
        out_shape=[jax.ShapeDtypeStruct((n, k), F32),
                   jax.ShapeDtypeStruct((n, k), F32)],
    )(x, Wl, bl.reshape(1, k), Wr, br.reshape(1, k))


def _tc_edge_scores(gl, gr, amat, eb):
    epad, k = gl.shape
    h = amat.shape[1]
    grid = (epad // eb,)

    def body(gl_ref, gr_ref, a_ref, e_ref, m_ref):
        z = gl_ref[...] + gr_ref[...]
        z = jnp.where(z > 0, z, 0.2 * z)
        e = jnp.dot(z, a_ref[...], preferred_element_type=F32)
        e_ref[...] = e
        bm = jnp.max(e, axis=0, keepdims=True)

        @pl.when(pl.program_id(0) == 0)
        def _():
            m_ref[...] = jnp.full((1, h), -jnp.inf, F32)

        m_ref[...] = jnp.maximum(m_ref[...], bm)

    return pl.pallas_call(
        body,
        grid=grid,
        in_specs=[
            pl.BlockSpec((eb, k), lambda i: (i, 0)),
            pl.BlockSpec((eb, k), lambda i: (i, 0)),
            pl.BlockSpec((k, h), lambda i: (0, 0)),
        ],
        out_specs=[pl.BlockSpec((eb, h), lambda i: (i, 0)),
                   pl.BlockSpec((1, h), lambda i: (0, 0))],
        out_shape=[jax.ShapeDtypeStruct((epad, h), F32),
                   jax.ShapeDtypeStruct((1, h), F32)],
    )(gl, gr, amat)


def _tc_exv(e, m, r16, eb):
    epad, h = e.shape
    grid = (epad // eb,)

    def body(e_ref, m_ref, r16_ref, x_ref):
        ex = jnp.exp(e_ref[...] - m_ref[...])
        x_ref[...] = jnp.dot(ex, r16_ref[...], preferred_element_type=F32)

    return pl.pallas_call(
        body,
        grid=grid,
        in_specs=[
            pl.BlockSpec((eb, h), lambda i: (i, 0)),
            pl.BlockSpec((1, h), lambda i: (0, 0)),
            pl.BlockSpec((h, 16), lambda i: (0, 0)),
        ],
        out_specs=pl.BlockSpec((eb, 16), lambda i: (i, 0)),
        out_shape=jax.ShapeDtypeStruct((epad, 16), F32),
    )(e, m, r16)


def _tc_norm_elu_transform(acc, s, bias, rep, W2l, b2l, W2r, b2r, nb):
    npad, k = acc.shape
    h4 = rep.shape[0]
    d2 = W2l.shape[1]
    grid = (npad // nb,)

    def body(acc_ref, s_ref, bias_ref, rep_ref, wl_ref, bl_ref, wr_ref, br_ref,
             ol_ref, or_ref):
        s4 = s_ref[...][:, :h4]
        recip = 1.0 / (s4 + 1e-16)
        hv = acc_ref[...] * jnp.dot(recip, rep_ref[...], preferred_element_type=F32)
        hv = hv + bias_ref[...]
        hv = jnp.where(hv > 0, hv, jnp.exp(hv) - 1.0)
        ol_ref[...] = jnp.dot(hv, wl_ref[...], preferred_element_type=F32) + bl_ref[...]
        or_ref[...] = jnp.dot(hv, wr_ref[...], preferred_element_type=F32) + br_ref[...]

    return pl.pallas_call(
        body,
        grid=grid,
        in_specs=[
            pl.BlockSpec((nb, k), lambda i: (i, 0)),
            pl.BlockSpec((nb, 16), lambda i: (i, 0)),
            pl.BlockSpec((1, k), lambda i: (0, 0)),
            pl.BlockSpec((h4, k), lambda i: (0, 0)),
            pl.BlockSpec((k, d2), lambda i: (0, 0)),
            pl.BlockSpec((1, d2), lambda i: (0, 0)),
            pl.BlockSpec((k, d2), lambda i: (0, 0)),
            pl.BlockSpec((1, d2), lambda i: (0, 0)),
        ],
        out_specs=[pl.BlockSpec((nb, d2), lambda i: (i, 0)),
                   pl.BlockSpec((nb, d2), lambda i: (i, 0))],
        out_shape=[jax.ShapeDtypeStruct((npad, d2), F32),
                   jax.ShapeDtypeStruct((npad, d2), F32)],
    )(acc, s, bias.reshape(1, k), rep, W2l, b2l.reshape(1, d2), W2r,
      b2r.reshape(1, d2))


def _tc_final_pool(acc, s, bias, batch3, n, g, nb):
    npad, d = acc.shape
    grid = (n // nb,)
    nsteps = n // nb

    def body(acc_ref, s_ref, bias_ref, b_ref, out_ref, sums_ref, cnts_ref):
        i = pl.program_id(0)
        s0 = s_ref[...][:, :1]
        hv = acc_ref[...] / (s0 + 1e-16) + bias_ref[...]
        hv = jnp.where(hv > 0, hv, jnp.exp(hv) - 1.0)
        bq = b_ref[...].reshape(1, nb)
        gids = lax.broadcasted_iota(I32, (g, 1), 0)
        oh = (bq == gids).astype(F32)
        part = lax.dot_general(oh, hv, (((1,), (0,)), ((), ())),
                               preferred_element_type=F32)
        pc = jnp.sum(oh, axis=1, keepdims=True)

        @pl.when(i == 0)
        def _():
            sums_ref[...] = jnp.zeros_like(sums_ref)
            cnts_ref[...] = jnp.zeros_like(cnts_ref)

        sums_ref[...] += part
        cnts_ref[...] += jnp.broadcast_to(pc, cnts_ref.shape)

        @pl.when(i == nsteps - 1)
        def _():
            out_ref[...] = sums_ref[...] / jnp.maximum(cnts_ref[...], 1.0)

    return pl.pallas_call(
        body,
        grid=grid,
        in_specs=[
            pl.BlockSpec((nb, d), lambda i: (i, 0)),
            pl.BlockSpec((nb, 16), lambda i: (i, 0)),
            pl.BlockSpec((1, d), lambda i: (0, 0)),
            pl.BlockSpec((1, 1, nb), lambda i: (i, 0, 0)),
        ],
        out_specs=pl.BlockSpec((g, d), lambda i: (0, 0)),
        out_shape=jax.ShapeDtypeStruct((g, d), F32),
        scratch_shapes=[pltpu.VMEM((g, d), F32), pltpu.VMEM((g, d), F32)],
    )(acc, s, bias.reshape(1, d), batch3)


def _sc_gather(xl, xr, srcp, dstp):
    epad = srcp.shape[0]
    dw = xl.shape[1]
    span = epad // NTILES
    iters = span // CH
    mesh = plsc.VectorSubcoreMesh(core_axis_name="c", subcore_axis_name="s")

    @functools.partial(
        pl.kernel,
        out_type=(jax.ShapeDtypeStruct((epad, dw), F32),
                  jax.ShapeDtypeStruct((epad, dw), F32)),
        mesh=mesh,
        scratch_types=[
            pltpu.VMEM((CH,), I32), pltpu.VMEM((CH,), I32),
            pltpu.VMEM((CH, dw), F32), pltpu.VMEM((CH, dw), F32),
            pltpu.SemaphoreType.DMA, pltpu.SemaphoreType.DMA,
        ],
        compiler_params=pltpu.CompilerParams(use_tc_tiling_on_sc=False, needs_layout_passes=False),
    )
    def k(xl_h, xr_h, src_h, dst_h, gl_h, gr_h, sib, dib, rowl, rowr, sm1, sm2):
        c = lax.axis_index("c")
        t = lax.axis_index("s")
        wid = t * 2 + c

        def chunk(j, carry):
            base = wid * span + j * CH
            pltpu.sync_copy(src_h.at[pl.ds(base, CH)], sib)
            pltpu.sync_copy(dst_h.at[pl.ds(base, CH)], dib)
            cp1 = pltpu.async_copy(xl_h.at[sib], rowl, sm1)
            cp2 = pltpu.async_copy(xr_h.at[dib], rowr, sm2)
            cp1.wait()
            cp2.wait()
            pltpu.sync_copy(rowl, gl_h.at[pl.ds(base, CH)])
            pltpu.sync_copy(rowr, gr_h.at[pl.ds(base, CH)])
            return carry

        lax.fori_loop(0, iters, chunk, 0)

    return k(xl, xr, srcp, dstp)


def _sc_bucket(srcp, dstp, e_real, rng):
    epad = srcp.shape[0]
    span = epad // NTILES
    iters = span // CH
    cap = span + CH
    mesh = plsc.VectorSubcoreMesh(core_axis_name="c", subcore_axis_name="s")

    @functools.partial(
        pl.kernel,
        out_type=(jax.ShapeDtypeStruct((NTILES, 4, cap), I32),
                  jax.ShapeDtypeStruct((NTILES, 4, cap), I32),
                  jax.ShapeDtypeStruct((NTILES, 4, cap), I32),
                  jax.ShapeDtypeStruct((NTILES, 16), I32)),
        mesh=mesh,
        scratch_types=[pltpu.VMEM((CH,), I32), pltpu.VMEM((CH,), I32)]
                      + [pltpu.VMEM((272,), I32)] * 12
                      + [pltpu.VMEM((16,), I32)],
        compiler_params=pltpu.CompilerParams(use_tc_tiling_on_sc=False, needs_layout_passes=False),
    )
    def k(src_h, dst_h, eid_h, srco_h, rel_h, cnt_h, sbuf, dbuf, *stg_cnt):
        stge, stgs, stgr = stg_cnt[0:4], stg_cnt[4:8], stg_cnt[8:12]
        cntb = stg_cnt[12]
        c = lax.axis_index("c")
        t = lax.axis_index("s")
        wid = t * 2 + c
        iota = lax.broadcasted_iota(I32, (16,), 0)

        def flush(b, cur, hch):
            def do(args):
                cur, hch = args
                for stg, out in ((stge[b], eid_h), (stgs[b], srco_h),
                                 (stgr[b], rel_h)):
                    pltpu.sync_copy(stg.at[pl.ds(0, CH)],
                                    out.at[wid, b, pl.ds(hch * CH, CH)])
                    for tm in range(8):
                        stg[pl.ds(tm * 16, 16)] = stg[pl.ds(CH + tm * 16, 16)]
                return (cur - CH, hch + 1)

            return lax.cond(cur >= CH, do, lambda a: a, (cur, hch))

        def chunk(j, carry):
            curs, hchs = carry
            base = wid * span + j * CH
            pltpu.sync_copy(src_h.at[pl.ds(base, CH)], sbuf)
            pltpu.sync_copy(dst_h.at[pl.ds(base, CH)], dbuf)
            new_curs, new_hchs = list(curs), list(hchs)
            for gi in range(CH // 16):
                d = dbuf[pl.ds(gi * 16, 16)]
                s = sbuf[pl.ds(gi * 16, 16)]
                eid = base + gi * 16 + iota
                pad = eid >= e_real
                bv = ((d >= rng).astype(I32) + (d >= 2 * rng).astype(I32)
                      + (d >= 3 * rng).astype(I32))
                for b in range(4):
                    m = (bv == b) & jnp.logical_not(pad)
                    cnt = lax.reduce_sum(m.astype(I32), axes=(0,))
                    cur = new_curs[b]
                    plsc.store_compressed(stge[b].at[pl.ds(cur, 16)], eid,
                                          mask=m)
                    plsc.store_compressed(stgs[b].at[pl.ds(cur, 16)], s,
                                          mask=m)
                    plsc.store_compressed(stgr[b].at[pl.ds(cur, 16)],
                                          d - b * rng, mask=m)
                    new_curs[b] = cur + cnt
            for b in range(4):
                new_curs[b], new_hchs[b] = flush(b, new_curs[b], new_hchs[b])
            return (tuple(new_curs), tuple(new_hchs))

        zero = jnp.zeros((), I32)
        curs, hchs = lax.fori_loop(0, iters, chunk,
                                   ((zero,) * 4, (zero,) * 4))

        trash_e = jnp.zeros((16,), I32)
        trash_r = jnp.full((16,), rng, I32)
        final = []
        for b in range(4):
            cur, hch = curs[b], hchs[b]
            for kp in range(8):
                stge[b][pl.ds(cur + kp * 16, 16)] = trash_e
                stgs[b][pl.ds(cur + kp * 16, 16)] = trash_e
                stgr[b][pl.ds(cur + kp * 16, 16)] = trash_r
            for stg, out in ((stge[b], eid_h), (stgs[b], srco_h),
                             (stgr[b], rel_h)):
                pltpu.sync_copy(stg.at[pl.ds(0, CH)],
                                out.at[wid, b, pl.ds(hch * CH, CH)])
            final.append(hch + 1)
        cv = jnp.zeros((16,), I32)
        for b in range(4):
            cv = jnp.where(iota == b, final[b], cv)
        cntb[...] = cv
        pltpu.sync_copy(cntb, cnt_h.at[wid])

    return k(srcp, dstp)


def _sc_scatter_bucketed(table, exv, lists, cnt, z_a, rng, nheads, quarter):
    eid_h, srco_h, rel_h = lists
    dw = table.shape[1] if table is not None else 16
    dsub = dw // nheads if nheads else 0
    srows = (rng if quarter else 2 * rng) + 16
    npad = 4 * rng
    za = srows // 16
    ca = (srows - 16) // 16
    mesh = plsc.VectorSubcoreMesh(core_axis_name="c", subcore_axis_name="s")

    scratch = [
        pltpu.VMEM_SHARED((srows, dw), F32),
        pltpu.VMEM((CH,), I32), pltpu.VMEM((CH,), I32),
        pltpu.VMEM((CH,), I32), pltpu.VMEM((CH,), I32),
        pltpu.VMEM((CH, 16), F32),
        pltpu.VMEM((2, 16), I32),
        pltpu.SemaphoreType.DMA, pltpu.SemaphoreType.DMA,
    ]
    if table is not None:
        scratch.append(pltpu.VMEM((CH, dw), F32))

    def body(*refs):
        if table is not None:
            (tab_h, ex_h, eid2, src2, rel2, cnt2, za_h, acc_h,
             slab, eidb, srcb, relb, idxb, exb, cntb, sm1, sm2, rowb) = refs
        else:
            (ex_h, eid2, src2, rel2, cnt2, za_h, acc_h,
             slab, eidb, srcb, relb, idxb, exb, cntb, sm1, sm2) = refs
        c = lax.axis_index("c")
        t = lax.axis_index("s")
        it16 = lax.broadcasted_iota(I32, (16,), 0)
        pltpu.sync_copy(cnt2.at[pl.ds(2 * t, 2)], cntb)

        def process(q, b2):
            for poff in range(2):
                p = 2 * t + poff
                cntv = cntb[poff]
                nch = lax.reduce_max(jnp.where(it16 == q, cntv, 0), axes=(0,))

                def chunk(j, carry):
                    off = j * CH
                    pltpu.sync_copy(eid2.at[p, q, pl.ds(off, CH)], eidb)
                    cpe = pltpu.async_copy(ex_h.at[eidb], exb, sm2)
                    if table is not None:
                        pltpu.sync_copy(src2.at[p, q, pl.ds(off, CH)], srcb)
                        cpr = pltpu.async_copy(tab_h.at[srcb], rowb, sm1)
                    pltpu.sync_copy(rel2.at[p, q, pl.ds(off, CH)], relb)
                    if b2 is None:
                        idx_ref = relb
                    else:
                        for gi in range(CH // 16):
                            rel = relb[pl.ds(gi * 16, 16)]
                            idxb[pl.ds(gi * 16, 16)] = jnp.where(
                                rel >= rng, 2 * rng, rel + b2 * rng)
                        idx_ref = idxb
                    cpe.wait()
                    if table is not None:
                        cpr.wait()

                        def edge(kk, c2):
                            kkv = jnp.broadcast_to(kk, (16,))
                            for h in range(nheads):
                                hv = jnp.full((16,), h, I32)
                                mult = plsc.load_gather(exb, [kkv, hv])
                                for jj in range(dsub // 16):
                                    col = h * dsub + jj * 16
                                    rowb[kk, pl.ds(col, 16)] = (
                                        rowb[kk, pl.ds(col, 16)] * mult)
                            return c2

                        lax.fori_loop(0, CH, edge, 0)
                        pltpu.sync_copy(rowb, slab.at[idx_ref], add=True)
                    else:
                        pltpu.sync_copy(exb, slab.at[idx_ref], add=True)
                    return carry

                lax.fori_loop(0, nch, chunk, 0)

        if quarter:
            for r in range(2):
                pltpu.sync_copy(za_h.at[pl.ds(0, za)],
                                slab.at[pl.ds(t * za, za)])
                plsc.subcore_barrier()
                process(2 * c + r, None)
                plsc.subcore_barrier()
                pltpu.sync_copy(
                    slab.at[pl.ds(t * ca, ca)],
                    acc_h.at[pl.ds((2 * c + r) * rng + t * ca, ca)])
                if r == 0:
                    plsc.subcore_barrier()
        else:
            pltpu.sync_copy(za_h.at[pl.ds(0, za)], slab.at[pl.ds(t * za, za)])
            plsc.subcore_barrier()
            for b2 in range(2):
                process(2 * c + b2, b2)
            plsc.subcore_barrier()
            pltpu.sync_copy(slab.at[pl.ds(t * ca, ca)],
                            acc_h.at[pl.ds(c * 2 * rng + t * ca, ca)])

    args = ([table] if table is not None else []) + [exv, eid_h, srco_h,
                                                     rel_h, cnt, z_a]
    kfn = functools.partial(
        pl.kernel,
        out_type=jax.ShapeDtypeStruct((npad, dw), F32),
        mesh=mesh,
        scratch_types=scratch,
        compiler_params=pltpu.CompilerParams(use_tc_tiling_on_sc=False, needs_layout_passes=False),
    )(body)
    return kfn(*args)


def kernel(x, W1l, b1l, W1r, b1r, att1, bias1, W2l, b2l, W2r, b2r, att2,
           bias2, edge_index, batch):
    n, f_in = x.shape
    e = edge_index.shape[1]
    h1, d = att1.shape
    hd = h1 * d
    g = 64

    epad = _cdiv(e, NTILES * CH) * NTILES * CH
    rng1 = _cdiv(n, 4 * 256) * 256
    half = 2 * rng1
    npad = 2 * half

    src = edge_index[0]
    dst = edge_index[1]
    padlen = epad - e
    srcp = jnp.concatenate([src, jnp.zeros((padlen,), I32)])
    dstp = jnp.concatenate([dst, jnp.zeros((padlen,), I32)])
    batch3 = batch.reshape(25, 1, n // 25)

    a1 = jnp.einsum("hd,hk->hdk", att1, jnp.eye(h1, dtype=F32)).reshape(hd, h1)
    rep1 = jnp.kron(jnp.eye(h1, dtype=F32), jnp.ones((1, d), F32))
    r16a = jnp.eye(h1, 16, dtype=F32)
    a2 = jnp.zeros((d, 8), F32).at[:, 0].set(att2[0])
    r16b = jnp.zeros((8, 16), F32).at[0, 0].set(1.0)

    z_a1 = jnp.zeros(((rng1 + 16) // 16, hd), F32)
    z_s1 = jnp.zeros(((half + 16) // 16, 16), F32)
    z_a2 = jnp.zeros(((half + 16) // 16, d), F32)

    bk_eid, bk_src, bk_rel, bk_cnt = _sc_bucket(srcp, dstp, e, rng1)
    lists = (bk_eid, bk_src, bk_rel)

    xl1, xr1 = _tc_node_transform(x, W1l, b1l, W1r, b1r, nb=2000)
    gl1, gr1 = _sc_gather(xl1, xr1, srcp, dstp)
    e1, m1 = _tc_edge_scores(gl1, gr1, a1, eb=4096)
    exv1 = _tc_exv(e1, m1, r16a, eb=4096)
    acc1 = _sc_scatter_bucketed(xl1, exv1, lists, bk_cnt, z_a1, rng1,
                                nheads=4, quarter=True)
    s1 = _sc_scatter_bucketed(None, exv1, lists, bk_cnt, z_s1, rng1,
                              nheads=0, quarter=False)
    xl2, xr2 = _tc_norm_elu_transform(acc1, s1, bias1, rep1, W2l, b2l, W2r,
                                      b2r, nb=1792)

    gl2, gr2 = _sc_gather(xl2, xr2, srcp, dstp)
    e2, m2 = _tc_edge_scores(gl2, gr2, a2, eb=4096)
    exv2 = _tc_exv(e2, m2, r16b, eb=4096)
    acc2 = _sc_scatter_bucketed(xl2, exv2, lists, bk_cnt, z_a2, rng1,
                                nheads=1, quarter=False)
    s2 = _sc_scatter_bucketed(None, exv2, lists, bk_cnt, z_s1, rng1,
                              nheads=0, quarter=False)

    return _tc_final_pool(acc2, s2, bias2, batch3, n, g, nb=2000)
